# baseline (speedup 1.0000x reference)
import functools

import jax
from jax import lax
import jax.numpy as jnp
from jax.experimental import pallas as pl
from jax.experimental.pallas import tpu as pltpu
from jax.experimental.pallas import tpu_sc as plsc

D_MODEL = 64
SCALE = 8.0
LANES = 16
C = 128
NBUF = 4
NW = 32


def kernel(x, table):
    S, T = x.shape
    V, D = table.shape
    n_units = T * (S // C)
    per_w = n_units // NW
    n_groups = per_w // NBUF

    xt = x.T.astype(jnp.int32).reshape(n_units, C)

    mesh = plsc.VectorSubcoreMesh(core_axis_name="core", subcore_axis_name="subcore")

    @functools.partial(
        pl.kernel,
        out_type=jax.ShapeDtypeStruct((T, D // 8, S // C, 8 * C), table.dtype),
        mesh=mesh,
        scratch_types=[
            pltpu.VMEM((per_w, C), jnp.int32),
            pltpu.VMEM((NBUF, C, D_MODEL), jnp.float32),
            pltpu.VMEM((NBUF, D_MODEL * C), jnp.float32),
            pltpu.SemaphoreType.DMA,
            [pltpu.SemaphoreType.DMA] * NBUF,
            [pltpu.SemaphoreType.DMA] * NBUF,
        ],
        compiler_params=pltpu.CompilerParams(
            use_tc_tiling_on_sc=False, needs_layout_passes=False),
    )
    def gather_scale(table_hbm, idx_hbm, out_hbm, idx_v, in_v, out_v,
                     sem_i, sem_g, sem_o):
        wid = lax.axis_index("subcore") * 2 + lax.axis_index("core")
        u0 = wid * per_w
        pltpu.async_copy(idx_hbm.at[pl.ds(u0, per_w)], idx_v, sem_i).wait()

        def start_gather(step, b):
            pltpu.make_async_copy(
                table_hbm.at[idx_v.at[step]], in_v.at[b], sem_g[b]).start()

        def wait_gather(step, b):
            pltpu.make_async_copy(
                table_hbm.at[idx_v.at[step]], in_v.at[b], sem_g[b]).wait()

        def out_copies(step, b):
            u = u0 + step
            t = u // (S // C)
            tg = u % (S // C)
            return [
                pltpu.make_async_copy(
                    out_v.at[b, pl.ds(j * 8 * C, 8 * C)], out_hbm.at[t, j, tg],
                    sem_o[b])
                for j in range(D // 8)
            ]

        rows = [
            lax.iota(jnp.int32, LANES) + sg * LANES
            for sg in range(C // LANES)
        ]

        def transpose_scale(b):
            src = in_v.at[b]

            @pl.loop(0, D_MODEL, step=4)
            def _(d0):
                for dd in range(4):
                    d = d0 + dd
                    col = jnp.full((LANES,), 0, jnp.int32) + d
                    for sg in range(C // LANES):
                        vals = plsc.load_gather(src, [rows[sg], col])
                        out_v[b, pl.ds(d * C + sg * LANES, LANES)] = vals * SCALE

        for b in range(NBUF):
            start_gather(b, b)

        @pl.loop(0, n_groups)
        def _(g):
            step0 = g * NBUF
            for b in range(NBUF):
                wait_gather(step0 + b, b)

                @pl.when(g > 0)
                def _():
                    for cp in out_copies(step0 + b - NBUF, b):
                        cp.wait()

                transpose_scale(b)

                for cp in out_copies(step0 + b, b):
                    cp.start()

                @pl.when(g < n_groups - 1)
                def _():
                    start_gather(step0 + b + NBUF, b)

        for b in range(NBUF):
            for cp in out_copies(per_w - NBUF + b, b):
                cp.wait()

    out4d = gather_scale(table, xt)
    out5d = out4d.reshape(T, D // 8, S // C, 8, C)
    return out5d.transpose(2, 4, 0, 1, 3).reshape(S, T, D)


# speedup vs baseline: 1.1323x; 1.1323x over previous
import functools

import jax
from jax import lax
import jax.numpy as jnp
from jax.experimental import pallas as pl
from jax.experimental.pallas import tpu as pltpu
from jax.experimental.pallas import tpu_sc as plsc

D_MODEL = 64
SCALE = 8.0
LANES = 16
C = 128
NBUF = 4
NW = 32


def kernel(x, table):
    S, T = x.shape
    V, D = table.shape
    n_units = T * (S // C)
    per_w = n_units // NW
    n_groups = per_w // NBUF

    xt = x.T.astype(jnp.int32).reshape(n_units, C)

    mesh = plsc.VectorSubcoreMesh(core_axis_name="core", subcore_axis_name="subcore")

    @functools.partial(
        pl.kernel,
        out_type=jax.ShapeDtypeStruct((T, D // 8, S // C, 8 * C), table.dtype),
        mesh=mesh,
        scratch_types=[
            pltpu.VMEM((per_w, C), jnp.int32),
            pltpu.VMEM((NBUF, C, D_MODEL), jnp.float32),
            pltpu.VMEM((NBUF, D_MODEL * C), jnp.float32),
            pltpu.SemaphoreType.DMA,
            [pltpu.SemaphoreType.DMA] * NBUF,
            [pltpu.SemaphoreType.DMA] * NBUF,
        ],
        compiler_params=pltpu.CompilerParams(
            use_tc_tiling_on_sc=False, needs_layout_passes=False),
    )
    def gather_scale(table_hbm, idx_hbm, out_hbm, idx_v, in_v, out_v,
                     sem_i, sem_g, sem_o):
        wid = lax.axis_index("subcore") * 2 + lax.axis_index("core")
        u0 = wid * per_w
        pltpu.async_copy(idx_hbm.at[pl.ds(u0, per_w)], idx_v, sem_i).wait()

        def start_gather(step, b):
            pltpu.make_async_copy(
                table_hbm.at[idx_v.at[step]], in_v.at[b], sem_g[b]).start()

        def wait_gather(step, b):
            pltpu.make_async_copy(
                table_hbm.at[idx_v.at[step]], in_v.at[b], sem_g[b]).wait()

        def out_copies(step, b):
            u = u0 + step
            t = u // (S // C)
            tg = u % (S // C)
            return [
                pltpu.make_async_copy(
                    out_v.at[b, pl.ds(j * 8 * C, 8 * C)], out_hbm.at[t, j, tg],
                    sem_o[b])
                for j in range(D // 8)
            ]

        stride_c = lax.iota(jnp.int32, LANES) * C

        def transpose_scale(b):
            src = in_v.at[b]
            dst = out_v.at[b]

            @pl.loop(0, C, step=4)
            def _(s0):
                for ss in range(4):
                    s = s0 + ss
                    for dc in range(D_MODEL // LANES):
                        vals = src[s, pl.ds(dc * LANES, LANES)] * SCALE
                        idx = stride_c + (s + dc * LANES * C)
                        plsc.store_scatter(dst, [idx], vals)

        for b in range(NBUF):
            start_gather(b, b)

        @pl.loop(0, n_groups)
        def _(g):
            step0 = g * NBUF
            for b in range(NBUF):
                wait_gather(step0 + b, b)

                @pl.when(g > 0)
                def _():
                    for cp in out_copies(step0 + b - NBUF, b):
                        cp.wait()

                transpose_scale(b)

                for cp in out_copies(step0 + b, b):
                    cp.start()

                @pl.when(g < n_groups - 1)
                def _():
                    start_gather(step0 + b + NBUF, b)

        for b in range(NBUF):
            for cp in out_copies(per_w - NBUF + b, b):
                cp.wait()

    out4d = gather_scale(table, xt)
    out5d = out4d.reshape(T, D // 8, S // C, 8, C)
    return out5d.transpose(2, 4, 0, 1, 3).reshape(S, T, D)


# speedup vs baseline: 1.4828x; 1.3096x over previous
import functools

import jax
from jax import lax
import jax.numpy as jnp
from jax.experimental import pallas as pl
from jax.experimental.pallas import tpu as pltpu
from jax.experimental.pallas import tpu_sc as plsc

D_MODEL = 64
SCALE = 8.0
LANES = 16
C = 128
NBUF = 4
NW = 32


def kernel(x, table):
    S, T = x.shape
    V, D = table.shape
    n_units = T * (S // C)
    per_w = n_units // NW
    n_groups = per_w // NBUF

    xt = x.T.astype(jnp.int32).reshape(n_units, C)

    mesh = plsc.VectorSubcoreMesh(core_axis_name="core", subcore_axis_name="subcore")

    @functools.partial(
        pl.kernel,
        out_type=jax.ShapeDtypeStruct((T, D // 8, S // C, 8 * C), table.dtype),
        mesh=mesh,
        scratch_types=[
            pltpu.VMEM((per_w, C), jnp.int32),
            pltpu.VMEM((NBUF, C, D_MODEL), jnp.float32),
            pltpu.VMEM((NBUF, D_MODEL * C), jnp.float32),
            pltpu.SemaphoreType.DMA,
            [pltpu.SemaphoreType.DMA] * NBUF,
            [pltpu.SemaphoreType.DMA] * NBUF,
        ],
        compiler_params=pltpu.CompilerParams(
            use_tc_tiling_on_sc=False, needs_layout_passes=False),
    )
    def gather_scale(table_hbm, idx_hbm, out_hbm, idx_v, in_v, out_v,
                     sem_i, sem_g, sem_o):
        wid = lax.axis_index("subcore") * 2 + lax.axis_index("core")
        u0 = wid * per_w
        pltpu.async_copy(idx_hbm.at[pl.ds(u0, per_w)], idx_v, sem_i).wait()

        def start_gather(step, b):
            pltpu.make_async_copy(
                table_hbm.at[idx_v.at[step]], in_v.at[b], sem_g[b]).start()

        def wait_gather(step, b):
            pltpu.make_async_copy(
                table_hbm.at[idx_v.at[step]], in_v.at[b], sem_g[b]).wait()

        def out_copies(step, b):
            u = u0 + step
            t = u // (S // C)
            tg = u % (S // C)
            return [
                pltpu.make_async_copy(
                    out_v.at[b, pl.ds(j * 8 * C, 8 * C)], out_hbm.at[t, j, tg],
                    sem_o[b])
                for j in range(D // 8)
            ]

        stride_c = lax.iota(jnp.int32, LANES) * C

        def transpose_scale(b):
            src = in_v.at[b]
            dst = out_v.at[b]

            @plsc.parallel_loop(0, C, step=1, unroll=8)
            def _(s):
                for dc in range(D_MODEL // LANES):
                    vals = src[s, pl.ds(dc * LANES, LANES)] * SCALE
                    idx = stride_c + (s + dc * LANES * C)
                    plsc.store_scatter(dst, [idx], vals)

        for b in range(NBUF):
            start_gather(b, b)

        @pl.loop(0, n_groups)
        def _(g):
            step0 = g * NBUF
            for b in range(NBUF):
                wait_gather(step0 + b, b)

                @pl.when(g > 0)
                def _():
                    for cp in out_copies(step0 + b - NBUF, b):
                        cp.wait()

                transpose_scale(b)

                for cp in out_copies(step0 + b, b):
                    cp.start()

                @pl.when(g < n_groups - 1)
                def _():
                    start_gather(step0 + b + NBUF, b)

        for b in range(NBUF):
            for cp in out_copies(per_w - NBUF + b, b):
                cp.wait()

    out4d = gather_scale(table, xt)
    out5d = out4d.reshape(T, D // 8, S // C, 8, C)
    return out5d.transpose(2, 4, 0, 1, 3).reshape(S, T, D)


# speedup vs baseline: 1.4924x; 1.0064x over previous
import functools

import jax
from jax import lax
import jax.numpy as jnp
from jax.experimental import pallas as pl
from jax.experimental.pallas import tpu as pltpu
from jax.experimental.pallas import tpu_sc as plsc

D_MODEL = 64
SCALE = 8.0
LANES = 16
C = 128
NBUF = 4
NW = 32


def kernel(x, table):
    S, T = x.shape
    V, D = table.shape
    n_units = T * (S // C)
    per_w = n_units // NW
    n_groups = per_w // NBUF

    xt = x.T.astype(jnp.int32).reshape(n_units, C)

    mesh = plsc.VectorSubcoreMesh(core_axis_name="core", subcore_axis_name="subcore")

    @functools.partial(
        pl.kernel,
        out_type=jax.ShapeDtypeStruct((T, D // 8, S // C, 8 * C), table.dtype),
        mesh=mesh,
        scratch_types=[
            pltpu.VMEM((per_w, C), jnp.int32),
            pltpu.VMEM((NBUF, C, D_MODEL), jnp.float32),
            pltpu.VMEM((NBUF, D // 8, 8 * C), jnp.float32),
            pltpu.SemaphoreType.DMA,
            [pltpu.SemaphoreType.DMA] * NBUF,
            [pltpu.SemaphoreType.DMA] * NBUF,
        ],
        compiler_params=pltpu.CompilerParams(
            use_tc_tiling_on_sc=False, needs_layout_passes=False),
    )
    def gather_scale(table_hbm, idx_hbm, out_hbm, idx_v, in_v, out_v,
                     sem_i, sem_g, sem_o):
        wid = lax.axis_index("subcore") * 2 + lax.axis_index("core")
        u0 = wid * per_w
        pltpu.async_copy(idx_hbm.at[pl.ds(u0, per_w)], idx_v, sem_i).wait()

        def start_gather(step, b):
            pltpu.make_async_copy(
                table_hbm.at[idx_v.at[step]], in_v.at[b], sem_g[b]).start()

        def wait_gather(step, b):
            pltpu.make_async_copy(
                table_hbm.at[idx_v.at[step]], in_v.at[b], sem_g[b]).wait()

        def out_copies(step, b):
            u = u0 + step
            t = lax.shift_right_logical(u, 5)
            tg = lax.bitwise_and(u, (S // C) - 1)
            return [
                pltpu.make_async_copy(
                    out_v.at[b], out_hbm.at[t, :, tg], sem_o[b])
            ]

        lane = lax.iota(jnp.int32, LANES)

        def transpose_scale(b):
            src = in_v.at[b]
            dst = out_v.at[b]
            rows = [
                lax.shift_right_logical(lane + dc * LANES, 3)
                for dc in range(D_MODEL // LANES)
            ]
            cols0 = [
                lax.bitwise_and(lane + dc * LANES, 7) * C
                for dc in range(D_MODEL // LANES)
            ]

            @plsc.parallel_loop(0, C, step=1, unroll=8)
            def _(s):
                for dc in range(D_MODEL // LANES):
                    vals = src[s, pl.ds(dc * LANES, LANES)] * SCALE
                    plsc.store_scatter(dst, [rows[dc], cols0[dc] + s], vals)

        for b in range(NBUF):
            start_gather(b, b)

        @pl.loop(0, n_groups)
        def _(g):
            step0 = g * NBUF
            for b in range(NBUF):
                wait_gather(step0 + b, b)

                @pl.when(g > 0)
                def _():
                    for cp in out_copies(step0 + b - NBUF, b):
                        cp.wait()

                transpose_scale(b)

                for cp in out_copies(step0 + b, b):
                    cp.start()

                @pl.when(g < n_groups - 1)
                def _():
                    start_gather(step0 + b + NBUF, b)

        for b in range(NBUF):
            for cp in out_copies(per_w - NBUF + b, b):
                cp.wait()

    out4d = gather_scale(table, xt)
    out5d = out4d.reshape(T, D // 8, S // C, 8, C)
    return out5d.transpose(2, 4, 0, 1, 3).reshape(S, T, D)


# speedup vs baseline: 2.5872x; 1.7337x over previous
import functools

import jax
from jax import lax
import jax.numpy as jnp
from jax.experimental import pallas as pl
from jax.experimental.pallas import tpu as pltpu
from jax.experimental.pallas import tpu_sc as plsc

D_MODEL = 64
SCALE = 8.0
LANES = 16
C = 128
NBUF = 4
NW = 32


def kernel(x, table):
    S, T = x.shape
    V, D = table.shape
    n_units = T * (S // C)
    per_w = n_units // NW
    n_groups = per_w // NBUF

    xt = x.T.astype(jnp.int32).reshape(n_units, C)

    mesh = plsc.VectorSubcoreMesh(core_axis_name="core", subcore_axis_name="subcore")

    @functools.partial(
        pl.kernel,
        out_type=jax.ShapeDtypeStruct((T, D // 8, S // C, 8, C), table.dtype),
        mesh=mesh,
        scratch_types=[
            pltpu.VMEM((per_w, C), jnp.int32),
            pltpu.VMEM((NBUF, C, D_MODEL), jnp.float32),
            pltpu.VMEM((NBUF, D // 8, 8, C + 1), jnp.float32),
            pltpu.SemaphoreType.DMA,
            [pltpu.SemaphoreType.DMA] * NBUF,
            [pltpu.SemaphoreType.DMA] * NBUF,
        ],
        compiler_params=pltpu.CompilerParams(
            use_tc_tiling_on_sc=False, needs_layout_passes=False),
    )
    def gather_scale(table_hbm, idx_hbm, out_hbm, idx_v, in_v, out_v,
                     sem_i, sem_g, sem_o):
        wid = lax.axis_index("subcore") * 2 + lax.axis_index("core")
        u0 = wid * per_w
        pltpu.async_copy(idx_hbm.at[pl.ds(u0, per_w)], idx_v, sem_i).wait()

        def start_gather(step, b):
            pltpu.make_async_copy(
                table_hbm.at[idx_v.at[step]], in_v.at[b], sem_g[b]).start()

        def wait_gather(step, b):
            pltpu.make_async_copy(
                table_hbm.at[idx_v.at[step]], in_v.at[b], sem_g[b]).wait()

        def out_copies(step, b):
            u = u0 + step
            t = lax.shift_right_logical(u, 5)
            tg = lax.bitwise_and(u, (S // C) - 1)
            return [
                pltpu.make_async_copy(
                    out_v.at[b, :, :, pl.ds(0, C)], out_hbm.at[t, :, tg],
                    sem_o[b])
            ]

        lane = lax.iota(jnp.int32, LANES)

        def transpose_scale(b):
            src = in_v.at[b]
            dst = out_v.at[b]
            rows_j = [
                lax.shift_right_logical(lane + dc * LANES, 3)
                for dc in range(D_MODEL // LANES)
            ]
            rows_di = [
                lax.bitwise_and(lane + dc * LANES, 7)
                for dc in range(D_MODEL // LANES)
            ]

            @plsc.parallel_loop(0, C, step=1, unroll=8)
            def _(s):
                col = jnp.full((LANES,), 0, jnp.int32) + s
                for dc in range(D_MODEL // LANES):
                    vals = src[s, pl.ds(dc * LANES, LANES)] * SCALE
                    plsc.store_scatter(dst, [rows_j[dc], rows_di[dc], col], vals)

        for b in range(NBUF):
            start_gather(b, b)

        @pl.loop(0, n_groups)
        def _(g):
            step0 = g * NBUF
            for b in range(NBUF):
                wait_gather(step0 + b, b)

                @pl.when(g > 0)
                def _():
                    for cp in out_copies(step0 + b - NBUF, b):
                        cp.wait()

                transpose_scale(b)

                for cp in out_copies(step0 + b, b):
                    cp.start()

                @pl.when(g < n_groups - 1)
                def _():
                    start_gather(step0 + b + NBUF, b)

        for b in range(NBUF):
            for cp in out_copies(per_w - NBUF + b, b):
                cp.wait()

    out5d = gather_scale(table, xt)
    return out5d.transpose(2, 4, 0, 1, 3).reshape(S, T, D)
